# baseline (speedup 1.0000x reference)
import functools

import jax
import jax.numpy as jnp
from jax import lax
from jax.experimental import pallas as pl
from jax.experimental.pallas import tpu as pltpu
from jax.experimental.pallas import tpu_sc as plsc

D = 256
N0, N1, N2 = 5, 6, 2
ROWS = N0 * N1 * N2
SP = 64
NC, NS = 2, 16
NW = NC * NS
CHUNK = 128
L = 16


@jax.jit
def _bond_encode_sc(ea0, ea1, ea2, w0, w1, w2):
    E = ea0.shape[0]
    assert E % NW == 0
    epw = E // NW
    nch = -(-epw // CHUNK)
    tail = epw - (nch - 1) * CHUNK
    epad = nch * CHUNK
    assert tail % 8 == 0 and nch % 2 == 0 and epw % 8 == 0

    mesh = plsc.VectorSubcoreMesh(
        core_axis_name="c", subcore_axis_name="s",
        num_cores=NC, num_subcores=NS)

    @functools.partial(
        pl.kernel,
        mesh=mesh,
        out_type=jax.ShapeDtypeStruct((E * D,), jnp.float32),
        scratch_types=[
            pltpu.VMEM((N0 * D,), jnp.float32),
            pltpu.VMEM((N1 * D,), jnp.float32),
            pltpu.VMEM((N2 * D,), jnp.float32),
            pltpu.VMEM((SP * D,), jnp.float32),
            pltpu.VMEM((epad,), jnp.int32),
            pltpu.VMEM((epad,), jnp.int32),
            pltpu.VMEM((epad,), jnp.int32),
            pltpu.VMEM((epad,), jnp.int32),
            pltpu.VMEM((CHUNK * D,), jnp.float32),
            pltpu.VMEM((CHUNK * D,), jnp.float32),
            pltpu.SemaphoreType.DMA,
            pltpu.SemaphoreType.DMA,
            pltpu.SemaphoreType.DMA,
        ],
    )
    def k(ea0_h, ea1_h, ea2_h, w0_h, w1_h, w2_h, out_h,
          w0_v, w1_v, w2_v, s_v, i0_v, i1_v, i2_v, cidx_v,
          rows_a, rows_b, sem_in, sem_a, sem_b):
        cid = lax.axis_index("c")
        sid = lax.axis_index("s")
        wid = cid * NS + sid
        base = wid * epw

        pltpu.async_copy(ea0_h.at[pl.ds(base, epw)], i0_v.at[pl.ds(0, epw)],
                         sem_in)
        pltpu.async_copy(ea1_h.at[pl.ds(base, epw)], i1_v.at[pl.ds(0, epw)],
                         sem_in)
        pltpu.async_copy(ea2_h.at[pl.ds(base, epw)], i2_v.at[pl.ds(0, epw)],
                         sem_in)

        pltpu.sync_copy(w0_h, w0_v)
        pltpu.sync_copy(w1_h, w1_v)
        pltpu.sync_copy(w2_h, w2_v)

        def build_row(c, carry):
            i0 = c // (N1 * N2)
            r = c - i0 * (N1 * N2)
            i1 = r // N2
            i2 = r - i1 * N2
            b_s, b0, b1, b2 = c * D, i0 * D, i1 * D, i2 * D
            for j in range(D // L):
                s_v[pl.ds(b_s + j * L, L)] = (
                    w0_v[pl.ds(b0 + j * L, L)]
                    + w1_v[pl.ds(b1 + j * L, L)]
                    + w2_v[pl.ds(b2 + j * L, L)])
            return carry

        lax.fori_loop(0, ROWS, build_row, None)

        for _ in range(3):
            pltpu.make_async_copy(
                ea0_h.at[pl.ds(0, epw)], i0_v.at[pl.ds(0, epw)], sem_in
            ).wait()

        def cidx_body(g, carry):
            sl = pl.ds(g * L, L)
            c = i0_v[sl] * (N1 * N2) + i1_v[sl] * N2 + i2_v[sl]
            cidx_v[sl] = jnp.minimum(jnp.maximum(c, 0), ROWS - 1) * D
            return carry

        lax.fori_loop(0, epad // L, cidx_body, None)

        def chunk_work(kk, buf, semb):
            @pl.when(kk >= 2)
            def _wait_prev():
                pltpu.make_async_copy(
                    buf, out_h.at[pl.ds(0, CHUNK * D)], semb).wait()

            def grp(g, carry):
                cvec = cidx_v[pl.ds(kk * CHUNK + g * L, L)]
                eb = g * (L * D)
                for l in range(L):
                    c = cvec[l]
                    for j in range(D // L):
                        buf[pl.ds(eb + l * D + j * L, L)] = (
                            s_v[pl.ds(c + j * L, L)])
                return carry

            lax.fori_loop(0, CHUNK // L, grp, None)
            be = (base + kk * CHUNK) * D

            @pl.when(kk < nch - 1)
            def _full():
                pltpu.async_copy(buf, out_h.at[pl.ds(be, CHUNK * D)], semb)

            @pl.when(kk == nch - 1)
            def _tail():
                pltpu.async_copy(buf.at[pl.ds(0, tail * D)],
                                 out_h.at[pl.ds(be, tail * D)], semb)

        def pair_body(t, carry):
            chunk_work(2 * t, rows_a, sem_a)
            chunk_work(2 * t + 1, rows_b, sem_b)
            return carry

        lax.fori_loop(0, nch // 2, pair_body, None)

        pltpu.make_async_copy(
            rows_a, out_h.at[pl.ds(0, CHUNK * D)], sem_a).wait()
        pltpu.make_async_copy(
            rows_b.at[pl.ds(0, tail * D)], out_h.at[pl.ds(0, tail * D)], sem_b
        ).wait()

    return k(ea0, ea1, ea2, w0, w1, w2)


def kernel(edge_attr, W0, W1, W2):
    E = edge_attr.shape[0]
    ea = edge_attr.astype(jnp.int32)
    out = _bond_encode_sc(ea[:, 0], ea[:, 1], ea[:, 2],
                          W0.reshape(-1), W1.reshape(-1), W2.reshape(-1))
    return out.reshape(E, D)


# speedup vs baseline: 5.6340x; 5.6340x over previous
import functools

import jax
import jax.numpy as jnp
from jax import lax
from jax.experimental import pallas as pl
from jax.experimental.pallas import tpu as pltpu
from jax.experimental.pallas import tpu_sc as plsc

D = 256
N0, N1, N2 = 5, 6, 2
ROWS = N0 * N1 * N2
SP = 64
NC, NS = 2, 16
NW = NC * NS
L = 16
DRAIN = 40


@jax.jit
def _bond_encode_sc(ea0, ea1, ea2, w0, w1, w2):
    E = ea0.shape[0]
    assert E % NW == 0
    epw = E // NW
    ngrp = epw // L
    tail = epw - ngrp * L
    epad = (ngrp + (1 if tail else 0)) * L
    assert epw % 8 == 0 and epw % DRAIN == 0

    mesh = plsc.VectorSubcoreMesh(
        core_axis_name="c", subcore_axis_name="s",
        num_cores=NC, num_subcores=NS)

    @functools.partial(
        pl.kernel,
        mesh=mesh,
        out_type=jax.ShapeDtypeStruct((E, D), jnp.float32),
        scratch_types=[
            pltpu.VMEM((N0, D), jnp.float32),
            pltpu.VMEM((N1, D), jnp.float32),
            pltpu.VMEM((N2, D), jnp.float32),
            pltpu.VMEM((SP, D), jnp.float32),
            pltpu.VMEM((epad,), jnp.int32),
            pltpu.VMEM((epad,), jnp.int32),
            pltpu.VMEM((epad,), jnp.int32),
            pltpu.VMEM((epad,), jnp.int32),
            pltpu.SemaphoreType.DMA,
            pltpu.SemaphoreType.DMA,
        ],
    )
    def k(ea0_h, ea1_h, ea2_h, w0_h, w1_h, w2_h, out_h,
          w0_v, w1_v, w2_v, s_v, i0_v, i1_v, i2_v, cidx_v, sem_in, sem_out):
        cid = lax.axis_index("c")
        sid = lax.axis_index("s")
        wid = cid * NS + sid
        base = wid * epw

        pltpu.async_copy(ea0_h.at[pl.ds(base, epw)], i0_v.at[pl.ds(0, epw)],
                         sem_in)
        pltpu.async_copy(ea1_h.at[pl.ds(base, epw)], i1_v.at[pl.ds(0, epw)],
                         sem_in)
        pltpu.async_copy(ea2_h.at[pl.ds(base, epw)], i2_v.at[pl.ds(0, epw)],
                         sem_in)

        pltpu.sync_copy(w0_h, w0_v)
        pltpu.sync_copy(w1_h, w1_v)
        pltpu.sync_copy(w2_h, w2_v)

        def build_row(c, carry):
            i0 = c // (N1 * N2)
            r = c - i0 * (N1 * N2)
            i1 = r // N2
            i2 = r - i1 * N2
            for j in range(D // L):
                sl = pl.ds(j * L, L)
                s_v[c, sl] = w0_v[i0, sl] + w1_v[i1, sl] + w2_v[i2, sl]
            return carry

        lax.fori_loop(0, ROWS, build_row, None)

        for _ in range(3):
            pltpu.make_async_copy(
                ea0_h.at[pl.ds(0, epw)], i0_v.at[pl.ds(0, epw)], sem_in
            ).wait()

        def cidx_body(g, carry):
            sl = pl.ds(g * L, L)
            c = i0_v[sl] * (N1 * N2) + i1_v[sl] * N2 + i2_v[sl]
            cidx_v[sl] = jnp.minimum(jnp.maximum(c, 0), ROWS - 1)
            return carry

        lax.fori_loop(0, epad // L, cidx_body, None)

        def grp(g, carry):
            cvec = cidx_v[pl.ds(g * L, L)]
            eb = base + g * L
            for l in range(L):
                pltpu.async_copy(s_v.at[cvec[l]], out_h.at[eb + l], sem_out)
            return carry

        lax.fori_loop(0, ngrp, grp, None)
        if tail:
            cvec = cidx_v[pl.ds(ngrp * L, L)]
            eb = base + ngrp * L
            for l in range(tail):
                pltpu.async_copy(s_v.at[cvec[l]], out_h.at[eb + l], sem_out)

        def drain(t, carry):
            pltpu.make_async_copy(s_v.at[pl.ds(0, DRAIN)],
                                  out_h.at[pl.ds(0, DRAIN)], sem_out).wait()
            return carry

        lax.fori_loop(0, epw // DRAIN, drain, None)

    return k(ea0, ea1, ea2, w0, w1, w2)


def kernel(edge_attr, W0, W1, W2):
    ea = edge_attr.astype(jnp.int32)
    return _bond_encode_sc(ea[:, 0], ea[:, 1], ea[:, 2], W0, W1, W2)
